# baseline (speedup 1.0000x reference)
import functools

import jax
import jax.numpy as jnp
from jax import lax
from jax.experimental import pallas as pl
from jax.experimental.pallas import tpu as pltpu
from jax.experimental.pallas import tpu_sc as plsc

_BB = 256
_TV = 2048

_NC, _NS = 2, 16
_NW = _NC * _NS


def _sc_gather(table, idx):
    V, D = table.shape
    B = idx.shape[0]
    b_per_w = B // _NW
    mesh = plsc.VectorSubcoreMesh(core_axis_name="c", subcore_axis_name="s")

    @functools.partial(
        pl.kernel,
        mesh=mesh,
        out_type=jax.ShapeDtypeStruct((B, D), table.dtype),
        scratch_types=[
            pltpu.VMEM((b_per_w,), jnp.int32),
            pltpu.VMEM((b_per_w, D), table.dtype),
            pltpu.SemaphoreType.DMA,
        ],
    )
    def gather_k(table_hbm, idx_hbm, out_hbm, idx_v, rows_v, sem):
        wid = lax.axis_index("s") * _NC + lax.axis_index("c")
        base = wid * b_per_w
        pltpu.sync_copy(idx_hbm.at[pl.ds(base, b_per_w)], idx_v)
        pltpu.async_copy(table_hbm.at[idx_v], rows_v, sem).wait()
        pltpu.sync_copy(rows_v, out_hbm.at[pl.ds(base, b_per_w)])

    return gather_k(table, idx)


def _pass1(cond, W1, b1r, alpha2, W2, b2r):
    B, CD = cond.shape
    H = W1.shape[0]
    V = W2.shape[0]
    NB = B // _BB
    NV = pl.cdiv(V, _TV)

    def body(cond_ref, w1_ref, b1_ref, a_ref, w2_ref, b2_ref,
             h_ref, m_ref, s_ref):
        j = pl.program_id(0)
        bi = pl.program_id(1)
        rows = pl.ds(bi * _BB, _BB)

        @pl.when(j == 0)
        def _():
            hx = lax.dot_general(cond_ref[...], w1_ref[...],
                                 (((1,), (1,)), ((), ())),
                                 preferred_element_type=jnp.float32)
            hx = hx + b1_ref[...]
            a = a_ref[0, 0]
            h_ref[rows, :] = jnp.where(hx >= 0, hx, a * hx)
            m_ref[rows, :] = jnp.full((_BB, 1), -1e30, jnp.float32)
            s_ref[rows, :] = jnp.zeros((_BB, 1), jnp.float32)

        logits = lax.dot_general(h_ref[rows, :], w2_ref[...],
                                 (((1,), (1,)), ((), ())),
                                 preferred_element_type=jnp.float32)
        logits = logits + b2_ref[...]
        col = j * _TV + lax.broadcasted_iota(jnp.int32, (1, _TV), 1)
        logits = jnp.where(col < V, logits, -1e30)

        m_old = m_ref[rows, :]
        m_new = jnp.maximum(m_old, jnp.max(logits, axis=1, keepdims=True))
        s_new = (s_ref[rows, :] * jnp.exp(m_old - m_new)
                 + jnp.sum(jnp.exp(logits - m_new), axis=1, keepdims=True))
        m_ref[rows, :] = m_new

        @pl.when(j < NV - 1)
        def _():
            s_ref[rows, :] = s_new

        @pl.when(j == NV - 1)
        def _():
            s_ref[rows, :] = 1.0 / s_new

    return pl.pallas_call(
        body,
        grid=(NV, NB),
        in_specs=[
            pl.BlockSpec((_BB, CD), lambda j, bi: (bi, 0)),
            pl.BlockSpec((H, CD), lambda j, bi: (0, 0)),
            pl.BlockSpec((1, H), lambda j, bi: (0, 0)),
            pl.BlockSpec((1, 1), lambda j, bi: (0, 0),
                         memory_space=pltpu.SMEM),
            pl.BlockSpec((_TV, H), lambda j, bi: (j, 0)),
            pl.BlockSpec((1, _TV), lambda j, bi: (0, j)),
        ],
        out_specs=[
            pl.BlockSpec((B, H), lambda j, bi: (0, 0)),
            pl.BlockSpec((B, 1), lambda j, bi: (0, 0)),
            pl.BlockSpec((B, 1), lambda j, bi: (0, 0)),
        ],
        out_shape=[
            jax.ShapeDtypeStruct((B, H), jnp.float32),
            jax.ShapeDtypeStruct((B, 1), jnp.float32),
            jax.ShapeDtypeStruct((B, 1), jnp.float32),
        ],
    )(cond, W1, b1r, alpha2, W2, b2r)


def _pass2(h, m, sinv, W2, b2r):
    B, H = h.shape
    V = W2.shape[0]
    NB = B // _BB
    NV = pl.cdiv(V, _TV)

    def body(h_ref, m_ref, s_ref, w2_ref, b2_ref, o_ref):
        bi = pl.program_id(1)
        rows = pl.ds(bi * _BB, _BB)
        logits = lax.dot_general(h_ref[rows, :], w2_ref[...],
                                 (((1,), (1,)), ((), ())),
                                 preferred_element_type=jnp.float32)
        logits = logits + b2_ref[...]
        o_ref[...] = jnp.exp(logits - m_ref[rows, :]) * s_ref[rows, :]

    return pl.pallas_call(
        body,
        grid=(NV, NB),
        in_specs=[
            pl.BlockSpec((B, H), lambda j, bi: (0, 0)),
            pl.BlockSpec((B, 1), lambda j, bi: (0, 0)),
            pl.BlockSpec((B, 1), lambda j, bi: (0, 0)),
            pl.BlockSpec((_TV, H), lambda j, bi: (j, 0)),
            pl.BlockSpec((1, _TV), lambda j, bi: (0, j)),
        ],
        out_specs=pl.BlockSpec((_BB, _TV), lambda j, bi: (bi, j)),
        out_shape=jax.ShapeDtypeStruct((B, V), jnp.float32),
    )(h, m, sinv, W2, b2r)


def kernel(word_0, word_1, embed, W1, b1, alpha, W2, b2):
    B = word_0.shape[0]
    V, D = embed.shape
    idx = jnp.concatenate([word_0[:, 0], word_1[:, 0]]).astype(jnp.int32)
    packed = embed.reshape(V // 2, 2 * D)
    g = _sc_gather(packed, idx // 2)
    e = jnp.where((idx % 2 == 1)[:, None], g[:, D:], g[:, :D])
    cond = e.reshape(B, 2 * D)
    h, m, sinv = _pass1(cond, W1, b1.reshape(1, -1),
                        alpha.reshape(1, 1), W2, b2.reshape(1, -1))
    return _pass2(h, m, sinv, W2, b2.reshape(1, -1))
